# baseline (speedup 1.0000x reference)
import functools

import jax
import jax.numpy as jnp
from jax.experimental import pallas as pl

VOCAB = 8192
HALF = VOCAB // 2
CHUNK = 256
D = 3
BETA = 0.25


def _vq_body(flat_ref, cbt_ref, cbtn_ref, cb_ref, q_ref, idx_ref, acc_ref):
    i = pl.program_id(0)
    f = flat_ref[...]
    f_bf = f.astype(jnp.bfloat16)
    fnorm = jnp.sum(f * f, axis=1, keepdims=True)
    t = f.shape[0]
    iota0 = jax.lax.broadcasted_iota(jnp.int32, (t, CHUNK), 1)

    halves = []
    for h in range(2):
        m = None
        for c in range(HALF // CHUNK):
            base = h * HALF + c * CHUNK
            sl = slice(base, base + CHUNK)
            cb_c = cbt_ref[:, sl]
            cnorm_c = jnp.sum(cb_c * cb_c, axis=0, keepdims=True)
            dotn = jax.lax.dot_general(
                f_bf, cbtn_ref[:, sl], (((1,), (0,)), ((), ())),
                preferred_element_type=jnp.float32)
            s = (fnorm + dotn) + cnorm_c
            mc = jnp.min(s, axis=1, keepdims=True)
            amc = jnp.min(jnp.where(s == mc, iota0, CHUNK),
                          axis=1, keepdims=True)
            onehot = (iota0 == amc).astype(jnp.float32)
            cand = jax.lax.dot_general(
                onehot, cb_ref[sl, :], (((1,), (0,)), ((), ())),
                preferred_element_type=jnp.float32,
                precision=jax.lax.Precision.HIGHEST)
            if m is None:
                m, am, vec = mc, amc + base, cand
            else:
                upd = mc < m
                m = jnp.where(upd, mc, m)
                am = jnp.where(upd, amc + base, am)
                vec = jnp.where(upd, cand, vec)
        halves.append((m, am, vec))

    (m1, am1, v1), (m2, am2, v2) = halves
    m1q = m1.astype(jnp.bfloat16).astype(jnp.float32)
    upd = m2 < m1q
    am = jnp.where(upd, am2, am1)
    quant = jnp.where(upd, v2, v1)
    idx_ref[...] = am
    diff = quant - f
    q_ref[...] = f + diff

    @pl.when(i == 0)
    def _():
        acc_ref[...] = jnp.zeros_like(acc_ref)

    acc_ref[...] += jnp.sum(diff * diff).reshape(1, 1)


@functools.partial(jax.jit, static_argnames=("block_t",))
def _vq(flat, codebook, block_t=256):
    n = flat.shape[0]
    grid = n // block_t
    cbt = codebook.T
    cbtn_bf = (-2.0 * cbt).astype(jnp.bfloat16)
    q, idx, acc = pl.pallas_call(
        _vq_body,
        grid=(grid,),
        in_specs=[
            pl.BlockSpec((block_t, D), lambda i: (i, 0)),
            pl.BlockSpec((D, VOCAB), lambda i: (0, 0)),
            pl.BlockSpec((D, VOCAB), lambda i: (0, 0)),
            pl.BlockSpec((VOCAB, D), lambda i: (0, 0)),
        ],
        out_specs=[
            pl.BlockSpec((block_t, D), lambda i: (i, 0)),
            pl.BlockSpec((block_t, 1), lambda i: (i, 0)),
            pl.BlockSpec((1, 1), lambda i: (0, 0)),
        ],
        out_shape=[
            jax.ShapeDtypeStruct((n, D), jnp.float32),
            jax.ShapeDtypeStruct((n, 1), jnp.int32),
            jax.ShapeDtypeStruct((1, 1), jnp.float32),
        ],
    )(flat, cbt, cbtn_bf, codebook)
    return q, idx, acc


def kernel(feats, codebook):
    b, l, d = feats.shape
    flat = feats.reshape(-1, d)
    q, idx, acc = _vq(flat, codebook)
    n = b * l
    vq_loss = (1.0 + BETA) * (acc[0, 0] / jnp.float32(n * d))
    return q.reshape(b, l, d), idx.reshape(b, l), vq_loss


# speedup vs baseline: 3.3626x; 3.3626x over previous
import functools

import jax
import jax.numpy as jnp
from jax.experimental import pallas as pl

VOCAB = 8192
HALF = VOCAB // 2
GROUPS = 64
LANES = 128
D = 3
BETA = 0.25


def _half_argmin(scores, iota, base):
    m = jnp.min(scores, axis=1, keepdims=True)
    am = jnp.min(jnp.where(scores == m, iota, VOCAB),
                 axis=1, keepdims=True)
    return m, am + base


def _vq_body(flat_ref, cbtn_ref, cbt_ref, cbr_ref, q_ref, idx_ref, acc_ref):
    i = pl.program_id(0)
    f = flat_ref[...]
    t = f.shape[0]
    fnorm = jnp.sum(f * f, axis=1, keepdims=True)
    cbt = cbt_ref[...]
    cnorm = jnp.sum(cbt * cbt, axis=0, keepdims=True)
    dotn = jax.lax.dot_general(
        f.astype(jnp.bfloat16), cbtn_ref[...],
        (((1,), (0,)), ((), ())),
        preferred_element_type=jnp.float32)
    scores = (fnorm + dotn) + cnorm
    iota = jax.lax.broadcasted_iota(jnp.int32, (t, HALF), 1)
    m1, am1 = _half_argmin(scores[:, :HALF], iota, 0)
    m2, am2 = _half_argmin(scores[:, HALF:], iota, HALF)
    m1q = m1.astype(jnp.bfloat16).astype(jnp.float32)
    upd = m2 < m1q
    am_i = jnp.where(upd, am2, am1)
    idx_ref[...] = am_i

    lo = jax.lax.rem(am_i, LANES)
    hi = jax.lax.div(am_i, LANES)
    iota_l = jax.lax.broadcasted_iota(jnp.int32, (t, LANES), 1)
    iota_g = jax.lax.broadcasted_iota(jnp.int32, (t, GROUPS), 1)
    oh_lo = (iota_l == lo).astype(jnp.float32)
    oh_hi = (iota_g == hi).astype(jnp.float32)
    z = jax.lax.dot_general(
        oh_lo, cbr_ref[...], (((1,), (0,)), ((), ())),
        preferred_element_type=jnp.float32,
        precision=jax.lax.Precision.HIGHEST)
    parts = [jnp.sum(z[:, d * GROUPS:(d + 1) * GROUPS] * oh_hi,
                     axis=1, keepdims=True) for d in range(D)]
    quant = jnp.concatenate(parts, axis=1)
    diff = quant - f
    q_ref[...] = f + diff

    @pl.when(i == 0)
    def _():
        acc_ref[...] = jnp.zeros_like(acc_ref)

    acc_ref[...] += jnp.sum(diff * diff).reshape(1, 1)


@functools.partial(jax.jit, static_argnames=("block_t",))
def _vq(flat, codebook, block_t=256):
    n = flat.shape[0]
    grid = n // block_t
    cbt = codebook.T
    cbtn_bf = (-2.0 * cbt).astype(jnp.bfloat16)
    cbr = codebook.reshape(GROUPS, LANES, D).transpose(1, 2, 0)
    cbr = cbr.reshape(LANES, D * GROUPS)
    q, idx, acc = pl.pallas_call(
        _vq_body,
        grid=(grid,),
        in_specs=[
            pl.BlockSpec((block_t, D), lambda i: (i, 0)),
            pl.BlockSpec((D, VOCAB), lambda i: (0, 0)),
            pl.BlockSpec((D, VOCAB), lambda i: (0, 0)),
            pl.BlockSpec((LANES, D * GROUPS), lambda i: (0, 0)),
        ],
        out_specs=[
            pl.BlockSpec((block_t, D), lambda i: (i, 0)),
            pl.BlockSpec((block_t, 1), lambda i: (i, 0)),
            pl.BlockSpec((1, 1), lambda i: (0, 0)),
        ],
        out_shape=[
            jax.ShapeDtypeStruct((n, D), jnp.float32),
            jax.ShapeDtypeStruct((n, 1), jnp.int32),
            jax.ShapeDtypeStruct((1, 1), jnp.float32),
        ],
    )(flat, cbtn_bf, cbt, cbr)
    return q, idx, acc


def kernel(feats, codebook):
    b, l, d = feats.shape
    flat = feats.reshape(-1, d)
    q, idx, acc = _vq(flat, codebook)
    n = b * l
    vq_loss = (1.0 + BETA) * (acc[0, 0] / jnp.float32(n * d))
    return q.reshape(b, l, d), idx.reshape(b, l), vq_loss


# speedup vs baseline: 3.5623x; 1.0594x over previous
import functools

import jax
import jax.numpy as jnp
from jax.experimental import pallas as pl

VOCAB = 8192
HALF = VOCAB // 2
GROUPS = 64
LANES = 128
D = 3
BETA = 0.25


def _half_argmin(scores, iota, base):
    m = jnp.min(scores, axis=1, keepdims=True)
    am = jnp.min(jnp.where(scores == m, iota, VOCAB),
                 axis=1, keepdims=True)
    return m, am + base


def _vq_body(flat_ref, cbtn_ref, cbt_ref, cbr_ref, q_ref, idx_ref, acc_ref):
    i = pl.program_id(0)
    f = flat_ref[...]
    t = f.shape[0]
    fnorm = jnp.sum(f * f, axis=1, keepdims=True)
    cbt = cbt_ref[...]
    cnorm = jnp.sum(cbt * cbt, axis=0, keepdims=True)
    dotn = jax.lax.dot_general(
        f.astype(jnp.bfloat16), cbtn_ref[...],
        (((1,), (0,)), ((), ())),
        preferred_element_type=jnp.float32)
    scores = (fnorm + dotn) + cnorm
    iota = jax.lax.broadcasted_iota(jnp.int32, (t, HALF), 1)
    m1, am1 = _half_argmin(scores[:, :HALF], iota, 0)
    m2, am2 = _half_argmin(scores[:, HALF:], iota, HALF)
    m1q = m1.astype(jnp.bfloat16).astype(jnp.float32)
    upd = m2 < m1q
    am_i = jnp.where(upd, am2, am1)
    idx_ref[...] = am_i

    lo = jax.lax.rem(am_i, LANES)
    hi = jax.lax.div(am_i, LANES)
    iota_l = jax.lax.broadcasted_iota(jnp.int32, (t, LANES), 1)
    iota_g = jax.lax.broadcasted_iota(jnp.int32, (t, GROUPS), 1)
    oh_lo = (iota_l == lo).astype(jnp.float32)
    oh_hi = (iota_g == hi).astype(jnp.float32)
    z = jax.lax.dot_general(
        oh_lo, cbr_ref[...], (((1,), (0,)), ((), ())),
        preferred_element_type=jnp.float32,
        precision=jax.lax.Precision.HIGHEST)
    parts = [jnp.sum(z[:, d * GROUPS:(d + 1) * GROUPS] * oh_hi,
                     axis=1, keepdims=True) for d in range(D)]
    quant = jnp.concatenate(parts, axis=1)
    diff = quant - f
    q_ref[...] = f + diff

    @pl.when(i == 0)
    def _():
        acc_ref[...] = jnp.zeros_like(acc_ref)

    acc_ref[...] += jnp.sum(diff * diff).reshape(1, 1)


@functools.partial(jax.jit, static_argnames=("block_t",))
def _vq(flat, codebook, block_t=512):
    n = flat.shape[0]
    grid = n // block_t
    cbt = codebook.T
    cbtn_bf = (-2.0 * cbt).astype(jnp.bfloat16)
    cbr = codebook.reshape(GROUPS, LANES, D).transpose(1, 2, 0)
    cbr = cbr.reshape(LANES, D * GROUPS)
    q, idx, acc = pl.pallas_call(
        _vq_body,
        grid=(grid,),
        in_specs=[
            pl.BlockSpec((block_t, D), lambda i: (i, 0)),
            pl.BlockSpec((D, VOCAB), lambda i: (0, 0)),
            pl.BlockSpec((D, VOCAB), lambda i: (0, 0)),
            pl.BlockSpec((LANES, D * GROUPS), lambda i: (0, 0)),
        ],
        out_specs=[
            pl.BlockSpec((block_t, D), lambda i: (i, 0)),
            pl.BlockSpec((block_t, 1), lambda i: (i, 0)),
            pl.BlockSpec((1, 1), lambda i: (0, 0)),
        ],
        out_shape=[
            jax.ShapeDtypeStruct((n, D), jnp.float32),
            jax.ShapeDtypeStruct((n, 1), jnp.int32),
            jax.ShapeDtypeStruct((1, 1), jnp.float32),
        ],
    )(flat, cbtn_bf, cbt, cbr)
    return q, idx, acc


def kernel(feats, codebook):
    b, l, d = feats.shape
    flat = feats.reshape(-1, d)
    q, idx, acc = _vq(flat, codebook)
    n = b * l
    vq_loss = (1.0 + BETA) * (acc[0, 0] / jnp.float32(n * d))
    return q.reshape(b, l, d), idx.reshape(b, l), vq_loss


# speedup vs baseline: 3.6299x; 1.0190x over previous
import functools

import jax
import jax.numpy as jnp
from jax.experimental import pallas as pl

VOCAB = 8192
HALF = VOCAB // 2
GROUPS = 64
LANES = 128
D = 3
BETA = 0.25


def _half_argmin(scores, iota, base):
    m = jnp.min(scores, axis=1, keepdims=True)
    am = jnp.min(jnp.where(scores == m, iota, VOCAB),
                 axis=1, keepdims=True)
    return m, am + base


def _vq_body(flat_ref, cbtn_ref, cbt_ref, cbr_ref, q_ref, idx_ref, acc_ref):
    i = pl.program_id(0)
    f = flat_ref[...]
    t = f.shape[0]
    fnorm = jnp.sum(f * f, axis=1, keepdims=True)
    cbt = cbt_ref[...]
    cnorm = jnp.sum(cbt * cbt, axis=0, keepdims=True)
    dotn = jax.lax.dot_general(
        f.astype(jnp.bfloat16), cbtn_ref[...],
        (((1,), (0,)), ((), ())),
        preferred_element_type=jnp.float32)
    scores = (fnorm + dotn) + cnorm
    iota = jax.lax.broadcasted_iota(jnp.int32, (t, HALF), 1)
    m1, am1 = _half_argmin(scores[:, :HALF], iota, 0)
    m2, am2 = _half_argmin(scores[:, HALF:], iota, HALF)
    m1q = m1.astype(jnp.bfloat16).astype(jnp.float32)
    upd = m2 < m1q
    am_i = jnp.where(upd, am2, am1)
    idx_ref[...] = am_i

    lo = jax.lax.rem(am_i, LANES)
    hi = jax.lax.div(am_i, LANES)
    iota_l = jax.lax.broadcasted_iota(jnp.int32, (t, LANES), 1)
    iota_g = jax.lax.broadcasted_iota(jnp.int32, (t, GROUPS), 1)
    oh_lo = (iota_l == lo).astype(jnp.float32)
    oh_hi = (iota_g == hi).astype(jnp.float32)
    z = jax.lax.dot_general(
        oh_lo, cbr_ref[...], (((1,), (0,)), ((), ())),
        preferred_element_type=jnp.float32,
        precision=jax.lax.Precision.HIGHEST)
    parts = [jnp.sum(z[:, d * GROUPS:(d + 1) * GROUPS] * oh_hi,
                     axis=1, keepdims=True) for d in range(D)]
    quant = jnp.concatenate(parts, axis=1)
    diff = quant - f
    q_ref[...] = f + diff

    @pl.when(i == 0)
    def _():
        acc_ref[...] = jnp.zeros_like(acc_ref)

    acc_ref[...] += jnp.sum(diff * diff).reshape(1, 1)


@functools.partial(jax.jit, static_argnames=("block_t",))
def _vq(flat, codebook, block_t=1024):
    n = flat.shape[0]
    grid = n // block_t
    cbt = codebook.T
    cbtn_bf = (-2.0 * cbt).astype(jnp.bfloat16)
    cbr = codebook.reshape(GROUPS, LANES, D).transpose(1, 2, 0)
    cbr = cbr.reshape(LANES, D * GROUPS)
    q, idx, acc = pl.pallas_call(
        _vq_body,
        grid=(grid,),
        in_specs=[
            pl.BlockSpec((block_t, D), lambda i: (i, 0)),
            pl.BlockSpec((D, VOCAB), lambda i: (0, 0)),
            pl.BlockSpec((D, VOCAB), lambda i: (0, 0)),
            pl.BlockSpec((LANES, D * GROUPS), lambda i: (0, 0)),
        ],
        out_specs=[
            pl.BlockSpec((block_t, D), lambda i: (i, 0)),
            pl.BlockSpec((block_t, 1), lambda i: (i, 0)),
            pl.BlockSpec((1, 1), lambda i: (0, 0)),
        ],
        out_shape=[
            jax.ShapeDtypeStruct((n, D), jnp.float32),
            jax.ShapeDtypeStruct((n, 1), jnp.int32),
            jax.ShapeDtypeStruct((1, 1), jnp.float32),
        ],
    )(flat, cbtn_bf, cbt, cbr)
    return q, idx, acc


def kernel(feats, codebook):
    b, l, d = feats.shape
    flat = feats.reshape(-1, d)
    q, idx, acc = _vq(flat, codebook)
    n = b * l
    vq_loss = (1.0 + BETA) * (acc[0, 0] / jnp.float32(n * d))
    return q.reshape(b, l, d), idx.reshape(b, l), vq_loss
